# baseline (speedup 1.0000x reference)
import functools

import jax
import jax.numpy as jnp
from jax import lax
from jax.experimental import pallas as pl
from jax.experimental.pallas import tpu as pltpu
from jax.experimental.pallas import tpu_sc as plsc

_PI = 3.141592653589793
_HIDDEN = 64
_GAMMA = 12.0
_EPSILON = 2.0
_EMB_RANGE = (_GAMMA + _EPSILON) / _HIDDEN
_PHASE_SCALE = _PI / _EMB_RANGE

_B = 16384
_POS = 1024
_NC = 2
_NS = 16
_NW = _NC * _NS
_BPW = _B // _NW
_CHUNK = 128
_NCHUNK = _BPW // _CHUNK

_TWO_OVER_PI = 0.6366197723675814
_PIO2_HI = 1.5707963705062866
_PIO2_LO = -4.371139000186241e-08
_S1, _S2, _S3 = -1.0 / 6.0, 1.0 / 120.0, -1.0 / 5040.0
_C1, _C2, _C3, _C4 = -0.5, 1.0 / 24.0, -1.0 / 720.0, 1.0 / 40320.0


_PREP_BLK = 2048


def _prep_body(x_ref, idx_ref, roff_ref):
    x = x_ref[...]
    y = jnp.transpose(x, (1, 0))
    h = y[0:1]
    r = y[1:2]
    t = y[2:3]
    idx_ref[...] = jnp.concatenate([h, t, r >> 1], axis=0)[:, None, :]
    roff_ref[...] = ((r & 1) << 6)[:, None, :]


def _prep(inp):
    return pl.pallas_call(
        _prep_body,
        grid=(_B // _PREP_BLK,),
        in_specs=[pl.BlockSpec((_PREP_BLK, 3), lambda i: (i, 0))],
        out_specs=[
            pl.BlockSpec((3, 1, _PREP_BLK), lambda i: (0, 0, i)),
            pl.BlockSpec((1, 1, _PREP_BLK), lambda i: (0, 0, i)),
        ],
        out_shape=[
            jax.ShapeDtypeStruct((3, 1, _B), jnp.int32),
            jax.ShapeDtypeStruct((1, 1, _B), jnp.int32),
        ],
    )(inp)


def _gather_ht_body(idx3, ent, out_h, out_t, hidx_v, tidx_v, rows_v, sem):
    wid = lax.axis_index("s") * _NC + lax.axis_index("c")
    base = wid * _BPW

    for j in range(_NCHUNK):
        sl = pl.ds(base + j * _CHUNK, _CHUNK)
        pltpu.sync_copy(idx3.at[0, 0, sl], hidx_v.at[j])
        pltpu.sync_copy(idx3.at[1, 0, sl], tidx_v.at[j])

    def gather_table(idx_v, table, dst):
        copies = [
            pltpu.async_copy(table.at[idx_v.at[j]],
                             dst.at[pl.ds(j * _CHUNK, _CHUNK)], sem)
            for j in range(_NCHUNK)
        ]
        for c in copies:
            c.wait()

    gather_table(hidx_v, ent, rows_v)
    pltpu.sync_copy(rows_v, out_h.at[pl.ds(base, _BPW)])
    gather_table(tidx_v, ent, rows_v)
    pltpu.sync_copy(rows_v, out_t.at[pl.ds(base, _BPW)])


def _gather_r_body(idx3, rel2, out_rp, ridx_v, rows_v, sem):
    wid = lax.axis_index("s") * _NC + lax.axis_index("c")
    base = wid * _BPW

    for j in range(_NCHUNK):
        sl = pl.ds(base + j * _CHUNK, _CHUNK)
        pltpu.sync_copy(idx3.at[2, 0, sl], ridx_v.at[j])
    copies = [
        pltpu.async_copy(rel2.at[ridx_v.at[j]],
                         rows_v.at[pl.ds(j * _CHUNK, _CHUNK)], sem)
        for j in range(_NCHUNK)
    ]
    for c in copies:
        c.wait()
    pltpu.sync_copy(rows_v, out_rp.at[pl.ds(base, _BPW)])


@functools.lru_cache(maxsize=1)
def _make_gathers():
  mesh = plsc.VectorSubcoreMesh(core_axis_name="c", subcore_axis_name="s")
  ht = functools.partial(
    pl.kernel,
    mesh=mesh,
    out_type=(
        jax.ShapeDtypeStruct((_B, 2 * _HIDDEN), jnp.float32),
        jax.ShapeDtypeStruct((_B, 2 * _HIDDEN), jnp.float32),
    ),
    scratch_types=[
        pltpu.VMEM((_NCHUNK, _CHUNK), jnp.int32),
        pltpu.VMEM((_NCHUNK, _CHUNK), jnp.int32),
        pltpu.VMEM((_BPW, 2 * _HIDDEN), jnp.float32),
        pltpu.SemaphoreType.DMA,
    ],
  )(_gather_ht_body)
  r = functools.partial(
    pl.kernel,
    mesh=mesh,
    out_type=jax.ShapeDtypeStruct((_B, 2 * _HIDDEN), jnp.float32),
    scratch_types=[
        pltpu.VMEM((_NCHUNK, _CHUNK), jnp.int32),
        pltpu.VMEM((_BPW, 2 * _HIDDEN), jnp.float32),
        pltpu.SemaphoreType.DMA,
    ],
  )(_gather_r_body)
  return ht, r


_MAGIC = 12582912.0


def _sincos(ph):
    t = ph * _TWO_OVER_PI + _MAGIC
    k = jax.lax.bitcast_convert_type(t, jnp.int32)
    kf = t - _MAGIC
    r = ph - kf * _PIO2_HI - kf * _PIO2_LO
    z = r * r
    s_r = r * (1.0 + z * (_S1 + z * (_S2 + z * _S3)))
    c_r = 1.0 + z * (_C1 + z * (_C2 + z * (_C3 + z * _C4)))
    swap = (k & 1) == 1
    sign_s = jnp.where((k & 2) == 2, -1.0, 1.0)
    sign_c = jnp.where(((k + 1) & 2) == 2, -1.0, 1.0)
    s = sign_s * jnp.where(swap, c_r, s_r)
    c = sign_c * jnp.where(swap, s_r, c_r)
    return s, c


_SCORE_BLK = 1024


def _score_body(h_ref, t_ref, rp_ref, roff_ref, p_ref, n_ref):
    i = pl.program_id(0)
    h = h_ref[...]
    t = t_ref[...]
    rp = rp_ref[...]
    par = roff_ref[...].reshape(_SCORE_BLK, 1) != 0
    r = jnp.where(par, rp[:, _HIDDEN:], rp[:, :_HIDDEN])
    s, c = _sincos(r * _PHASE_SCALE)
    c_full = jnp.concatenate([c, c], axis=1)
    s_full = jnp.concatenate([-s, s], axis=1)
    h_rot = pltpu.roll(h, _HIDDEN, 1)
    a = h * c_full + h_rot * s_full - t
    a2 = a * a
    v = jnp.sqrt(a2 + pltpu.roll(a2, _HIDDEN, 1))
    res = _GAMMA - 0.5 * jnp.sum(v, axis=1, keepdims=True)

    @pl.when(i == 0)
    def _():
        p_ref[...] = res

    @pl.when(i > 0)
    def _():
        n_ref[...] = res


def _score(h_rows, t_rows, rp_rows, roff):
    nblk = _B // _SCORE_BLK
    return pl.pallas_call(
        _score_body,
        grid=(nblk,),
        in_specs=[
            pl.BlockSpec((_SCORE_BLK, 2 * _HIDDEN), lambda i: (i, 0)),
            pl.BlockSpec((_SCORE_BLK, 2 * _HIDDEN), lambda i: (i, 0)),
            pl.BlockSpec((_SCORE_BLK, 2 * _HIDDEN), lambda i: (i, 0)),
            pl.BlockSpec((1, 1, _SCORE_BLK), lambda i: (0, 0, i)),
        ],
        out_specs=[
            pl.BlockSpec((_POS, 1), lambda i: (0, 0)),
            pl.BlockSpec((_SCORE_BLK, 1),
                         lambda i: (jnp.maximum(i - 1, 0), 0)),
        ],
        out_shape=[
            jax.ShapeDtypeStruct((_POS, 1), jnp.float32),
            jax.ShapeDtypeStruct((_B - _POS, 1), jnp.float32),
        ],
    )(h_rows, t_rows, rp_rows, roff)


def kernel(input, ent_emb, rel_emb):
    h_col = input[:, 0]
    r_col = input[:, 1]
    t_col = input[:, 2]
    idx3 = jnp.stack([h_col, t_col, r_col >> 1]).reshape(3, 1, _B)
    roff = ((r_col & 1) << 6).reshape(1, 1, _B)
    rel2 = rel_emb.reshape(rel_emb.shape[0] // 2, 2 * _HIDDEN)
    gather_ht, gather_r = _make_gathers()
    h_rows, t_rows = gather_ht(idx3, ent_emb)
    rp_rows = gather_r(idx3, rel2)
    p_score, n_score = _score(h_rows, t_rows, rp_rows, roff)
    return p_score, n_score


# speedup vs baseline: 1.0136x; 1.0136x over previous
import functools

import jax
import jax.numpy as jnp
from jax import lax
from jax.experimental import pallas as pl
from jax.experimental.pallas import tpu as pltpu
from jax.experimental.pallas import tpu_sc as plsc

_PI = 3.141592653589793
_HIDDEN = 64
_GAMMA = 12.0
_EPSILON = 2.0
_EMB_RANGE = (_GAMMA + _EPSILON) / _HIDDEN
_PHASE_SCALE = _PI / _EMB_RANGE

_B = 16384
_POS = 1024
_NC = 2
_NS = 16
_NW = _NC * _NS
_BPW = _B // _NW
_CHUNK = 128
_NCHUNK = _BPW // _CHUNK

_TWO_OVER_PI = 0.6366197723675814
_PIO2_HI = 1.5707963705062866
_PIO2_LO = -4.371139000186241e-08
_S1, _S2, _S3 = -1.0 / 6.0, 1.0 / 120.0, -1.0 / 5040.0
_C1, _C2, _C3, _C4 = -0.5, 1.0 / 24.0, -1.0 / 720.0, 1.0 / 40320.0


_PREP_BLK = 2048


def _prep_body(x_ref, idx_ref, roff_ref):
    x = x_ref[...]
    y = jnp.transpose(x, (1, 0))
    h = y[0:1]
    r = y[1:2]
    t = y[2:3]
    idx_ref[...] = jnp.concatenate([h, t, r >> 1], axis=0)[:, None, :]
    roff_ref[...] = ((r & 1) << 6)[:, None, :]


def _prep(inp):
    return pl.pallas_call(
        _prep_body,
        grid=(_B // _PREP_BLK,),
        in_specs=[pl.BlockSpec((_PREP_BLK, 3), lambda i: (i, 0))],
        out_specs=[
            pl.BlockSpec((3, 1, _PREP_BLK), lambda i: (0, 0, i)),
            pl.BlockSpec((1, 1, _PREP_BLK), lambda i: (0, 0, i)),
        ],
        out_shape=[
            jax.ShapeDtypeStruct((3, 1, _B), jnp.int32),
            jax.ShapeDtypeStruct((1, 1, _B), jnp.int32),
        ],
    )(inp)


def _gather_ht_body(idx3, ent, out_h, out_t, hidx_v, tidx_v, rows_v, sem):
    wid = lax.axis_index("s") * _NC + lax.axis_index("c")
    base = wid * _BPW

    for j in range(_NCHUNK):
        sl = pl.ds(base + j * _CHUNK, _CHUNK)
        pltpu.sync_copy(idx3.at[0, 0, sl], hidx_v.at[j])
        pltpu.sync_copy(idx3.at[1, 0, sl], tidx_v.at[j])

    def gather_table(idx_v, table, dst):
        copies = [
            pltpu.async_copy(table.at[idx_v.at[j]],
                             dst.at[pl.ds(j * _CHUNK, _CHUNK)], sem)
            for j in range(_NCHUNK)
        ]
        for c in copies:
            c.wait()

    gather_table(hidx_v, ent, rows_v)
    pltpu.sync_copy(rows_v, out_h.at[pl.ds(base, _BPW)])
    gather_table(tidx_v, ent, rows_v)
    pltpu.sync_copy(rows_v, out_t.at[pl.ds(base, _BPW)])


def _gather_r_body(idx3, rel2, out_rp, ridx_v, rows_v, sem):
    wid = lax.axis_index("s") * _NC + lax.axis_index("c")
    base = wid * _BPW

    for j in range(_NCHUNK):
        sl = pl.ds(base + j * _CHUNK, _CHUNK)
        pltpu.sync_copy(idx3.at[2, 0, sl], ridx_v.at[j])
    copies = [
        pltpu.async_copy(rel2.at[ridx_v.at[j]],
                         rows_v.at[pl.ds(j * _CHUNK, _CHUNK)], sem)
        for j in range(_NCHUNK)
    ]
    for c in copies:
        c.wait()
    pltpu.sync_copy(rows_v, out_rp.at[pl.ds(base, _BPW)])


@functools.lru_cache(maxsize=1)
def _make_gathers():
  mesh = plsc.VectorSubcoreMesh(core_axis_name="c", subcore_axis_name="s")
  ht = functools.partial(
    pl.kernel,
    mesh=mesh,
    out_type=(
        jax.ShapeDtypeStruct((_B, 2 * _HIDDEN), jnp.float32),
        jax.ShapeDtypeStruct((_B, 2 * _HIDDEN), jnp.float32),
    ),
    scratch_types=[
        pltpu.VMEM((_NCHUNK, _CHUNK), jnp.int32),
        pltpu.VMEM((_NCHUNK, _CHUNK), jnp.int32),
        pltpu.VMEM((_BPW, 2 * _HIDDEN), jnp.float32),
        pltpu.SemaphoreType.DMA,
    ],
  )(_gather_ht_body)
  r = functools.partial(
    pl.kernel,
    mesh=mesh,
    out_type=jax.ShapeDtypeStruct((_B, 2 * _HIDDEN), jnp.float32),
    scratch_types=[
        pltpu.VMEM((_NCHUNK, _CHUNK), jnp.int32),
        pltpu.VMEM((_BPW, 2 * _HIDDEN), jnp.float32),
        pltpu.SemaphoreType.DMA,
    ],
  )(_gather_r_body)
  return ht, r


_MAGIC = 12582912.0


def _sincos(ph):
    t = ph * _TWO_OVER_PI + _MAGIC
    k = jax.lax.bitcast_convert_type(t, jnp.int32)
    kf = t - _MAGIC
    r = ph - kf * _PIO2_HI - kf * _PIO2_LO
    z = r * r
    s_r = r * (1.0 + z * (_S1 + z * (_S2 + z * _S3)))
    c_r = 1.0 + z * (_C1 + z * (_C2 + z * (_C3 + z * _C4)))
    swap = (k & 1) == 1
    sign_s = jnp.where((k & 2) == 2, -1.0, 1.0)
    sign_c = jnp.where(((k + 1) & 2) == 2, -1.0, 1.0)
    s = sign_s * jnp.where(swap, c_r, s_r)
    c = sign_c * jnp.where(swap, s_r, c_r)
    return s, c


_SCORE_BLK = 1024


def _score_body(h_ref, t_ref, rp_ref, roff_ref, p_ref, n_ref):
    i = pl.program_id(0)
    h = h_ref[...]
    t = t_ref[...]
    rp = rp_ref[...]
    par = roff_ref[...].reshape(_SCORE_BLK, 1) != 0
    lane_hi = lax.broadcasted_iota(
        jnp.int32, (_SCORE_BLK, 2 * _HIDDEN), 1) >= _HIDDEN
    rr = jnp.where(jnp.logical_xor(par, lane_hi),
                   pltpu.roll(rp, _HIDDEN, 1), rp)
    s, c = _sincos(rr * _PHASE_SCALE)
    s_signed = jnp.where(lane_hi, s, -s)
    h_rot = pltpu.roll(h, _HIDDEN, 1)
    a = h * c + h_rot * s_signed - t
    a2 = a * a
    v = jnp.sqrt(a2 + pltpu.roll(a2, _HIDDEN, 1))
    res = _GAMMA - 0.5 * jnp.sum(v, axis=1, keepdims=True)

    @pl.when(i == 0)
    def _():
        p_ref[...] = res

    @pl.when(i > 0)
    def _():
        n_ref[...] = res


def _score(h_rows, t_rows, rp_rows, roff):
    nblk = _B // _SCORE_BLK
    return pl.pallas_call(
        _score_body,
        grid=(nblk,),
        in_specs=[
            pl.BlockSpec((_SCORE_BLK, 2 * _HIDDEN), lambda i: (i, 0)),
            pl.BlockSpec((_SCORE_BLK, 2 * _HIDDEN), lambda i: (i, 0)),
            pl.BlockSpec((_SCORE_BLK, 2 * _HIDDEN), lambda i: (i, 0)),
            pl.BlockSpec((1, 1, _SCORE_BLK), lambda i: (0, 0, i)),
        ],
        out_specs=[
            pl.BlockSpec((_POS, 1), lambda i: (0, 0)),
            pl.BlockSpec((_SCORE_BLK, 1),
                         lambda i: (jnp.maximum(i - 1, 0), 0)),
        ],
        out_shape=[
            jax.ShapeDtypeStruct((_POS, 1), jnp.float32),
            jax.ShapeDtypeStruct((_B - _POS, 1), jnp.float32),
        ],
    )(h_rows, t_rows, rp_rows, roff)


def kernel(input, ent_emb, rel_emb):
    rel2 = rel_emb.reshape(rel_emb.shape[0] // 2, 2 * _HIDDEN)
    h_col = input[:, 0]
    r_col = input[:, 1]
    t_col = input[:, 2]
    idx3 = jnp.stack([h_col, t_col, r_col >> 1]).reshape(3, 1, _B)
    roff = ((r_col & 1) << 6).reshape(1, 1, _B)
    gather_ht, gather_r = _make_gathers()
    h_rows, t_rows = gather_ht(idx3, ent_emb)
    rp_rows = gather_r(idx3, rel2)
    p_score, n_score = _score(h_rows, t_rows, rp_rows, roff)
    return p_score, n_score
